# baseline (speedup 1.0000x reference)
import functools

import jax
import jax.numpy as jnp
from jax import lax
from jax.experimental import pallas as pl
from jax.experimental.pallas import tpu as pltpu
from jax.experimental.pallas import tpu_sc as plsc

N = 10000
E = 320000
DIN = 128
D2 = 256
DOUT = 128

NC = 2
NS = 16
NW = NC * NS
K = 128
EPW = E // NW
CHF = EPW // K
TK = EPW - CHF * K
N_PAD = 10240
RPT = N_PAD // NS

_mesh = plsc.VectorSubcoreMesh(
    core_axis_name="c", subcore_axis_name="s", num_cores=NC, num_subcores=NS)


@functools.partial(
    pl.kernel,
    out_type=jax.ShapeDtypeStruct((NC, N_PAD), jnp.float32),
    mesh=_mesh,
    scratch_types=[
        pltpu.VMEM((K,), jnp.int32),
        pltpu.VMEM((K,), jnp.int32),
        pltpu.VMEM((K,), jnp.int32),
        pltpu.VMEM((K,), jnp.int32),
        pltpu.VMEM((TK,), jnp.int32),
        pltpu.VMEM((K,), jnp.float32),
        pltpu.VMEM((RPT,), jnp.float32),
        pltpu.VMEM_SHARED((N_PAD,), jnp.float32),
        pltpu.SemaphoreType.DMA,
        pltpu.SemaphoreType.DMA,
        pltpu.SemaphoreType.DMA,
        pltpu.SemaphoreType.DMA,
        pltpu.SemaphoreType.DMA,
        pltpu.SemaphoreType.DMA,
    ],
)
def _deg_kernel(dst_hbm, out_hbm, d0, d1, d2, d3, idx_t, ones_v, zbuf, acc,
                si0, si1, si2, si3, ss0, ss1):
    c = lax.axis_index("c")
    s = lax.axis_index("s")
    wid = s * NC + c
    r0 = s * RPT
    idb = (d0, d1, d2, d3)
    semi = (si0, si1, si2, si3)
    sems = (ss0, ss1)
    for j in range(K // 16):
        ones_v[pl.ds(j * 16, 16)] = jnp.ones((16,), jnp.float32)
    for j in range(RPT // 16):
        zbuf[pl.ds(j * 16, 16)] = jnp.zeros((16,), jnp.float32)
    base_e = wid * EPW
    pltpu.sync_copy(zbuf, acc.at[pl.ds(r0, RPT)])
    plsc.subcore_barrier()

    def start_idx(ci, ib):
        pltpu.async_copy(dst_hbm.at[pl.ds(base_e + ci * K, K)],
                         idb[ib], semi[ib])

    def wait_idx(ib):
        pltpu.make_async_copy(dst_hbm.at[pl.ds(0, K)], idb[ib],
                              semi[ib]).wait()

    def start_sc(ib, sb):
        pltpu.async_copy(ones_v, acc.at[idb[ib]], sems[sb], add=True)

    def wait_sc(sb):
        pltpu.make_async_copy(ones_v, acc.at[idb[0]], sems[sb]).wait()

    start_idx(0, 0)
    start_idx(1, 1)
    wait_idx(0)
    start_sc(0, 0)
    start_idx(2, 2)
    wait_idx(1)
    start_sc(1, 1)
    start_idx(3, 3)

    def group(g, carry):
        cc = 4 * g
        for j in range(4):
            ib = (j + 2) % 4
            sb = j % 2
            wait_idx(ib)
            wait_sc(sb)
            start_sc(ib, sb)
            start_idx(cc + j + 4, j)
        return carry

    lax.fori_loop(0, (CHF - 6) // 4, group, 0)
    cc = CHF - 6
    for j in range(4):
        ib = (j + 2) % 4
        sb = j % 2
        wait_idx(ib)
        wait_sc(sb)
        start_sc(ib, sb)
        if j < 2:
            start_idx(cc + j + 4, j)
    wait_sc(0)
    wait_sc(1)
    pltpu.sync_copy(dst_hbm.at[pl.ds(base_e + CHF * K, TK)], idx_t)
    pltpu.sync_copy(ones_v.at[pl.ds(0, TK)], acc.at[idx_t], add=True)
    plsc.subcore_barrier()
    pltpu.sync_copy(acc.at[pl.ds(r0, RPT)], out_hbm.at[c, pl.ds(r0, RPT)])


@functools.partial(
    pl.kernel,
    out_type=jax.ShapeDtypeStruct((NC, N_PAD, DIN), jnp.float32),
    mesh=_mesh,
    scratch_types=[
        pltpu.VMEM((K,), jnp.int32),
        pltpu.VMEM((K,), jnp.int32),
        pltpu.VMEM((K,), jnp.int32),
        pltpu.VMEM((K,), jnp.int32),
        pltpu.VMEM((K,), jnp.int32),
        pltpu.VMEM((K,), jnp.int32),
        pltpu.VMEM((K,), jnp.int32),
        pltpu.VMEM((K,), jnp.int32),
        pltpu.VMEM((TK,), jnp.int32),
        pltpu.VMEM((TK,), jnp.int32),
        pltpu.VMEM((K, DIN), jnp.float32),
        pltpu.VMEM((K, DIN), jnp.float32),
        pltpu.VMEM_SHARED((N_PAD, DIN), jnp.float32),
        pltpu.SemaphoreType.DMA,
        pltpu.SemaphoreType.DMA,
        pltpu.SemaphoreType.DMA,
        pltpu.SemaphoreType.DMA,
        pltpu.SemaphoreType.DMA,
        pltpu.SemaphoreType.DMA,
    ],
)
def _scatter_kernel(src_hbm, dst_hbm, y_hbm, out_hbm,
                    is0, is1, is2, is3, id0, id1, id2, id3, ist, idt,
                    rows0, rows1, acc,
                    si0, si1, si2, si3, sg0, sg1):
    c = lax.axis_index("c")
    s = lax.axis_index("s")
    wid = s * NC + c
    r0 = s * RPT
    isrc = (is0, is1, is2, is3)
    idst = (id0, id1, id2, id3)
    semi = (si0, si1, si2, si3)
    rows = (rows0, rows1)
    semg = (sg0, sg1)

    def zfill(r, carry):
        for j in range(DIN // 16):
            rows0[r, pl.ds(j * 16, 16)] = jnp.zeros((16,), jnp.float32)
        return carry

    lax.fori_loop(0, K, zfill, 0)
    for t in range(RPT // K):
        pltpu.sync_copy(rows0, acc.at[pl.ds(r0 + t * K, K), :])
    plsc.subcore_barrier()
    base_e = wid * EPW

    def start_idx(ci, ib):
        off = base_e + ci * K
        pltpu.async_copy(src_hbm.at[pl.ds(off, K)], isrc[ib], semi[ib])
        pltpu.async_copy(dst_hbm.at[pl.ds(off, K)], idst[ib], semi[ib])

    def wait_idx(ib):
        pltpu.make_async_copy(src_hbm.at[pl.ds(0, K)], isrc[ib],
                              semi[ib]).wait()
        pltpu.make_async_copy(dst_hbm.at[pl.ds(0, K)], idst[ib],
                              semi[ib]).wait()

    def start_gather(ib, rb):
        pltpu.async_copy(y_hbm.at[isrc[ib]], rows[rb], semg[rb])

    def wait_gather(rb):
        pltpu.make_async_copy(y_hbm.at[isrc[0]], rows[rb], semg[rb]).wait()

    def scatter_add(ib, rb):
        pltpu.sync_copy(rows[rb], acc.at[idst[ib]], add=True)

    start_idx(0, 0)
    start_idx(1, 1)
    wait_idx(0)
    start_gather(0, 0)
    start_idx(2, 2)
    wait_idx(1)
    start_gather(1, 1)
    wait_gather(0)
    scatter_add(0, 0)
    start_idx(3, 3)

    def group(g, carry):
        cc = 4 * g
        for j in range(4):
            ib = (j + 2) % 4
            rb = j % 2
            wait_idx(ib)
            start_gather(ib, rb)
            wait_gather(1 - rb)
            scatter_add((j + 1) % 4, 1 - rb)
            start_idx(cc + j + 4, j)
        return carry

    lax.fori_loop(0, (CHF - 6) // 4, group, 0)
    cc = CHF - 6
    for j in range(4):
        ib = (j + 2) % 4
        rb = j % 2
        wait_idx(ib)
        start_gather(ib, rb)
        wait_gather(1 - rb)
        scatter_add((j + 1) % 4, 1 - rb)
        if j < 2:
            start_idx(cc + j + 4, j)
    wait_gather(1)
    scatter_add(1, 1)
    pltpu.sync_copy(src_hbm.at[pl.ds(base_e + CHF * K, TK)], ist)
    pltpu.sync_copy(dst_hbm.at[pl.ds(base_e + CHF * K, TK)], idt)
    pltpu.async_copy(y_hbm.at[ist], rows0.at[pl.ds(0, TK), :], sg0).wait()
    pltpu.sync_copy(rows0.at[pl.ds(0, TK), :], acc.at[idt], add=True)
    plsc.subcore_barrier()
    pltpu.sync_copy(acc.at[pl.ds(r0, RPT), :],
                    out_hbm.at[c, pl.ds(r0, RPT), :])


def _prep_body(degp_ref, x_ref, y_ref, dinv_ref):
    d = degp_ref[0] + degp_ref[1] + 1.0
    dinv = lax.rsqrt(d)
    dinv_ref[...] = dinv
    y_ref[pl.ds(0, N), :] = x_ref[...] * dinv[0:N]
    y_ref[pl.ds(N, N_PAD - N), :] = jnp.zeros((N_PAD - N, DIN), jnp.float32)


def _prep(deg_partials, x):
    return pl.pallas_call(
        _prep_body,
        out_shape=(
            jax.ShapeDtypeStruct((N_PAD, DIN), jnp.float32),
            jax.ShapeDtypeStruct((N_PAD, 1), jnp.float32),
        ),
    )(deg_partials, x)


_NBLK = 16
_BR = N_PAD // _NBLK


def _final_body(p0_ref, p1_ref, x_ref, dinv_ref, w1_ref, b1_ref,
                f1w_ref, f1b_ref, f2w_ref, f2b_ref,
                f3w_ref, f3b_ref, f4w_ref, f4b_ref,
                o1_ref, o2_ref, o3_ref, o4_ref, gacc_ref):
    i = pl.program_id(0)
    dv = dinv_ref[...]
    sv = dv * (p0_ref[...] + p1_ref[...]) + (dv * dv) * x_ref[...]
    t = jnp.dot(sv, w1_ref[...], preferred_element_type=jnp.float32)
    t = jnp.maximum(t + b1_ref[...], 0.0)
    rows = i * _BR + lax.broadcasted_iota(jnp.int32, (_BR, 1), 0)
    t = jnp.where(rows < N, t, 0.0)

    @pl.when(i == 0)
    def _():
        gacc_ref[...] = jnp.zeros((1, D2), jnp.float32)

    gacc_ref[...] += jnp.sum(t, axis=0, keepdims=True)

    @pl.when(i == _NBLK - 1)
    def _():
        g = gacc_ref[...] * (1.0 / N)
        dn = (((1,), (1,)), ((), ()))
        o1_ref[...] = jnp.tanh(
            lax.dot_general(g, f1w_ref[...], dn,
                            preferred_element_type=jnp.float32)[0]
            + f1b_ref[...])
        o2_ref[...] = jnp.tanh(
            lax.dot_general(g, f2w_ref[...], dn,
                            preferred_element_type=jnp.float32)[0]
            + f2b_ref[...])
        o3_ref[...] = jnp.tanh(
            lax.dot_general(g, f3w_ref[...], dn,
                            preferred_element_type=jnp.float32)[0]
            + f3b_ref[...])
        o4_ref[...] = jnp.tanh(
            lax.dot_general(g, f4w_ref[...], dn,
                            preferred_element_type=jnp.float32)[0]
            + f4b_ref[...])


def _final(p0, p1, x, dinv, w1, b1, f1w, f1b, f2w, f2b,
           f3w, f3b, f4w, f4b):
    row_spec = pl.BlockSpec((_BR, DIN), lambda i: (i, 0))
    full = lambda shape: pl.BlockSpec(shape, lambda i: (0,) * len(shape))
    return pl.pallas_call(
        _final_body,
        grid=(_NBLK,),
        in_specs=[
            row_spec, row_spec,
            pl.BlockSpec((_BR, DIN), lambda i: (i, 0)),
            pl.BlockSpec((_BR, 1), lambda i: (i, 0)),
            full((DIN, D2)), full((D2,)),
            full((D2, D2)), full((D2,)),
            full((D2, D2)), full((D2,)),
            full((DOUT, D2)), full((DOUT,)),
            full((DOUT, D2)), full((DOUT,)),
        ],
        out_specs=[full((D2,)), full((D2,)), full((DOUT,)), full((DOUT,))],
        out_shape=(
            jax.ShapeDtypeStruct((D2,), jnp.float32),
            jax.ShapeDtypeStruct((D2,), jnp.float32),
            jax.ShapeDtypeStruct((DOUT,), jnp.float32),
            jax.ShapeDtypeStruct((DOUT,), jnp.float32),
        ),
        scratch_shapes=[pltpu.VMEM((1, D2), jnp.float32)],
    )(p0, p1, x, dinv, w1, b1, f1w, f1b, f2w, f2b, f3w, f3b, f4w, f4b)


def kernel(x, edge_index, conv1_weight, conv1_bias,
           fc1_weight, fc1_bias, fc2_weight, fc2_bias,
           fc3_weight, fc3_bias, fc4_weight, fc4_bias):
    srcv = edge_index[0]
    dstv = edge_index[1]
    deg_partials = _deg_kernel(dstv)
    y, dinv = _prep(deg_partials.reshape(NC, N_PAD, 1), x)
    partials = _scatter_kernel(srcv, dstv, y)
    return _final(partials[0], partials[1], x, dinv,
                  conv1_weight, conv1_bias, fc1_weight, fc1_bias,
                  fc2_weight, fc2_bias, fc3_weight, fc3_bias,
                  fc4_weight, fc4_bias)


# speedup vs baseline: 1.1899x; 1.1899x over previous
import functools

import jax
import jax.numpy as jnp
from jax import lax
from jax.experimental import pallas as pl
from jax.experimental.pallas import tpu as pltpu
from jax.experimental.pallas import tpu_sc as plsc

N = 10000
E = 320000
DIN = 128
D2 = 256
DOUT = 128

NC = 2
NS = 16
NW = NC * NS
K = 128
NCH = E // K
CHW = NCH // NW
NEX = NCH - CHW * NW
N_PAD = 10240
RPT = N_PAD // NS

_mesh = plsc.VectorSubcoreMesh(
    core_axis_name="c", subcore_axis_name="s", num_cores=NC, num_subcores=NS)


@functools.partial(
    pl.kernel,
    out_type=jax.ShapeDtypeStruct((NC, N_PAD), jnp.float32),
    mesh=_mesh,
    scratch_types=[
        pltpu.VMEM((2, K), jnp.int32),
        pltpu.VMEM((2, K), jnp.int32),
        pltpu.VMEM((2, K), jnp.int32),
        pltpu.VMEM((2, K), jnp.int32),
        pltpu.VMEM((K,), jnp.float32),
        pltpu.VMEM((RPT,), jnp.float32),
        pltpu.VMEM_SHARED((N_PAD,), jnp.float32),
        pltpu.SemaphoreType.DMA,
        pltpu.SemaphoreType.DMA,
        pltpu.SemaphoreType.DMA,
        pltpu.SemaphoreType.DMA,
        pltpu.SemaphoreType.DMA,
        pltpu.SemaphoreType.DMA,
    ],
)
def _deg_kernel(edge_hbm, out_hbm, e0, e1, e2, e3, ones_v, zbuf, acc,
                si0, si1, si2, si3, ss0, ss1):
    c = lax.axis_index("c")
    s = lax.axis_index("s")
    wid = s * NC + c
    r0 = s * RPT
    idb = (e0, e1, e2, e3)
    semi = (si0, si1, si2, si3)
    sems = (ss0, ss1)
    for j in range(K // 16):
        ones_v[pl.ds(j * 16, 16)] = jnp.ones((16,), jnp.float32)
    for j in range(RPT // 16):
        zbuf[pl.ds(j * 16, 16)] = jnp.zeros((16,), jnp.float32)
    pltpu.sync_copy(zbuf, acc.at[pl.ds(r0, RPT)])
    plsc.subcore_barrier()

    def start_idx(ci, ib):
        pltpu.async_copy(edge_hbm.at[:, pl.ds((wid + 32 * ci) * K, K)],
                         idb[ib], semi[ib])

    def wait_idx(ib):
        pltpu.make_async_copy(edge_hbm.at[:, pl.ds(0, K)], idb[ib],
                              semi[ib]).wait()

    def start_sc(ib, sb):
        pltpu.async_copy(ones_v, acc.at[idb[ib].at[1]], sems[sb], add=True)

    def wait_sc(sb):
        pltpu.make_async_copy(ones_v, acc.at[idb[0].at[1]], sems[sb]).wait()

    start_idx(0, 0)
    start_idx(1, 1)
    wait_idx(0)
    start_sc(0, 0)
    start_idx(2, 2)
    wait_idx(1)
    start_sc(1, 1)
    start_idx(3, 3)

    def group(g, carry):
        cc = 4 * g
        for j in range(4):
            ib = (j + 2) % 4
            sb = j % 2
            wait_idx(ib)
            wait_sc(sb)
            start_sc(ib, sb)
            start_idx(cc + j + 4, j)
        return carry

    lax.fori_loop(0, (CHW - 6) // 4, group, 0)
    cc = CHW - 6
    for j in range(4):
        ib = (j + 2) % 4
        sb = j % 2
        wait_idx(ib)
        wait_sc(sb)
        start_sc(ib, sb)
        if j < 2:
            start_idx(cc + j + 4, j)
    wait_sc(0)
    wait_sc(1)

    @pl.when(wid < NEX)
    def _():
        pltpu.sync_copy(edge_hbm.at[:, pl.ds((wid + 32 * CHW) * K, K)], e0)
        pltpu.sync_copy(ones_v, acc.at[e0.at[1]], add=True)

    plsc.subcore_barrier()
    pltpu.sync_copy(acc.at[pl.ds(r0, RPT)], out_hbm.at[c, pl.ds(r0, RPT)])


@functools.partial(
    pl.kernel,
    out_type=jax.ShapeDtypeStruct((NC, N_PAD, DIN), jnp.float32),
    mesh=_mesh,
    scratch_types=[
        pltpu.VMEM((2, K), jnp.int32),
        pltpu.VMEM((2, K), jnp.int32),
        pltpu.VMEM((2, K), jnp.int32),
        pltpu.VMEM((2, K), jnp.int32),
        pltpu.VMEM((K, DIN), jnp.float32),
        pltpu.VMEM((K, DIN), jnp.float32),
        pltpu.VMEM_SHARED((N_PAD, DIN), jnp.float32),
        pltpu.SemaphoreType.DMA,
        pltpu.SemaphoreType.DMA,
        pltpu.SemaphoreType.DMA,
        pltpu.SemaphoreType.DMA,
        pltpu.SemaphoreType.DMA,
        pltpu.SemaphoreType.DMA,
    ],
)
def _scatter_kernel(edge_hbm, y_hbm, out_hbm,
                    e0, e1, e2, e3, rows0, rows1, acc,
                    si0, si1, si2, si3, sg0, sg1):
    c = lax.axis_index("c")
    s = lax.axis_index("s")
    wid = s * NC + c
    r0 = s * RPT
    idb = (e0, e1, e2, e3)
    semi = (si0, si1, si2, si3)
    rows = (rows0, rows1)
    semg = (sg0, sg1)

    def zfill(r, carry):
        for j in range(DIN // 16):
            rows0[r, pl.ds(j * 16, 16)] = jnp.zeros((16,), jnp.float32)
        return carry

    lax.fori_loop(0, K, zfill, 0)
    for t in range(RPT // K):
        pltpu.sync_copy(rows0, acc.at[pl.ds(r0 + t * K, K), :])
    plsc.subcore_barrier()

    def start_idx(ci, ib):
        pltpu.async_copy(edge_hbm.at[:, pl.ds((wid + 32 * ci) * K, K)],
                         idb[ib], semi[ib])

    def wait_idx(ib):
        pltpu.make_async_copy(edge_hbm.at[:, pl.ds(0, K)], idb[ib],
                              semi[ib]).wait()

    def start_gather(ib, rb):
        pltpu.async_copy(y_hbm.at[idb[ib].at[0]], rows[rb], semg[rb])

    def wait_gather(rb):
        pltpu.make_async_copy(y_hbm.at[idb[0].at[0]], rows[rb],
                              semg[rb]).wait()

    def scatter_add(ib, rb):
        pltpu.sync_copy(rows[rb], acc.at[idb[ib].at[1]], add=True)

    start_idx(0, 0)
    start_idx(1, 1)
    wait_idx(0)
    start_gather(0, 0)
    start_idx(2, 2)
    wait_idx(1)
    start_gather(1, 1)
    wait_gather(0)
    scatter_add(0, 0)
    start_idx(3, 3)

    def group(g, carry):
        cc = 4 * g
        for j in range(4):
            ib = (j + 2) % 4
            rb = j % 2
            wait_idx(ib)
            start_gather(ib, rb)
            wait_gather(1 - rb)
            scatter_add((j + 1) % 4, 1 - rb)
            start_idx(cc + j + 4, j)
        return carry

    lax.fori_loop(0, (CHW - 6) // 4, group, 0)
    cc = CHW - 6
    for j in range(4):
        ib = (j + 2) % 4
        rb = j % 2
        wait_idx(ib)
        start_gather(ib, rb)
        wait_gather(1 - rb)
        scatter_add((j + 1) % 4, 1 - rb)
        if j < 2:
            start_idx(cc + j + 4, j)
    wait_gather(1)
    scatter_add(1, 1)

    @pl.when(wid < NEX)
    def _():
        pltpu.sync_copy(edge_hbm.at[:, pl.ds((wid + 32 * CHW) * K, K)], e0)
        pltpu.async_copy(y_hbm.at[e0.at[0]], rows0, sg0).wait()
        pltpu.sync_copy(rows0, acc.at[e0.at[1]], add=True)

    plsc.subcore_barrier()
    pltpu.sync_copy(acc.at[pl.ds(r0, RPT), :],
                    out_hbm.at[c, pl.ds(r0, RPT), :])


def _prep_body(degp_ref, x_ref, y_ref, dinv_ref):
    d1 = degp_ref[0] + degp_ref[1] + 1.0
    dinv1 = lax.rsqrt(d1)
    dinv_ref[...] = dinv1
    dcol = jnp.reshape(dinv1, (N_PAD, 1))
    y_ref[pl.ds(0, N), :] = x_ref[...] * dcol[0:N]
    y_ref[pl.ds(N, N_PAD - N), :] = jnp.zeros((N_PAD - N, DIN), jnp.float32)


def _prep(deg_partials, x):
    return pl.pallas_call(
        _prep_body,
        out_shape=(
            jax.ShapeDtypeStruct((N_PAD, DIN), jnp.float32),
            jax.ShapeDtypeStruct((N_PAD,), jnp.float32),
        ),
    )(deg_partials, x)


_NBLK = 16
_BR = N_PAD // _NBLK


def _final_body(p_ref, x_ref, dinv_ref, w1_ref, b1_ref,
                f1w_ref, f1b_ref, f2w_ref, f2b_ref,
                f3w_ref, f3b_ref, f4w_ref, f4b_ref,
                o1_ref, o2_ref, o3_ref, o4_ref, gacc_ref):
    i = pl.program_id(0)
    dv = jnp.reshape(dinv_ref[pl.ds(i * _BR, _BR)], (_BR, 1))
    sv = dv * (p_ref[0] + p_ref[1]) + (dv * dv) * x_ref[...]
    t = jnp.dot(sv, w1_ref[...], preferred_element_type=jnp.float32)
    t = jnp.maximum(t + b1_ref[...], 0.0)
    rows = i * _BR + lax.broadcasted_iota(jnp.int32, (_BR, 1), 0)
    t = jnp.where(rows < N, t, 0.0)

    @pl.when(i == 0)
    def _():
        gacc_ref[...] = jnp.zeros((1, D2), jnp.float32)

    gacc_ref[...] += jnp.sum(t, axis=0, keepdims=True)

    @pl.when(i == _NBLK - 1)
    def _():
        g = gacc_ref[...] * (1.0 / N)
        dn = (((1,), (1,)), ((), ()))
        o1_ref[...] = jnp.tanh(
            lax.dot_general(g, f1w_ref[...], dn,
                            preferred_element_type=jnp.float32)[0]
            + f1b_ref[...])
        o2_ref[...] = jnp.tanh(
            lax.dot_general(g, f2w_ref[...], dn,
                            preferred_element_type=jnp.float32)[0]
            + f2b_ref[...])
        o3_ref[...] = jnp.tanh(
            lax.dot_general(g, f3w_ref[...], dn,
                            preferred_element_type=jnp.float32)[0]
            + f3b_ref[...])
        o4_ref[...] = jnp.tanh(
            lax.dot_general(g, f4w_ref[...], dn,
                            preferred_element_type=jnp.float32)[0]
            + f4b_ref[...])


def _final(partials, x, dinv, w1, b1, f1w, f1b, f2w, f2b,
           f3w, f3b, f4w, f4b):
    full = lambda shape: pl.BlockSpec(shape, lambda i: (0,) * len(shape))
    return pl.pallas_call(
        _final_body,
        grid=(_NBLK,),
        in_specs=[
            pl.BlockSpec((NC, _BR, DIN), lambda i: (0, i, 0)),
            pl.BlockSpec((_BR, DIN), lambda i: (i, 0)),
            pl.BlockSpec((N_PAD,), lambda i: (0,)),
            full((DIN, D2)), full((D2,)),
            full((D2, D2)), full((D2,)),
            full((D2, D2)), full((D2,)),
            full((DOUT, D2)), full((DOUT,)),
            full((DOUT, D2)), full((DOUT,)),
        ],
        out_specs=[full((D2,)), full((D2,)), full((DOUT,)), full((DOUT,))],
        out_shape=(
            jax.ShapeDtypeStruct((D2,), jnp.float32),
            jax.ShapeDtypeStruct((D2,), jnp.float32),
            jax.ShapeDtypeStruct((DOUT,), jnp.float32),
            jax.ShapeDtypeStruct((DOUT,), jnp.float32),
        ),
        scratch_shapes=[pltpu.VMEM((1, D2), jnp.float32)],
    )(partials, x, dinv, w1, b1, f1w, f1b, f2w, f2b, f3w, f3b, f4w, f4b)


def kernel(x, edge_index, conv1_weight, conv1_bias,
           fc1_weight, fc1_bias, fc2_weight, fc2_bias,
           fc3_weight, fc3_bias, fc4_weight, fc4_bias):
    deg_partials = _deg_kernel(edge_index)
    y, dinv = _prep(deg_partials, x)
    partials = _scatter_kernel(edge_index, y)
    return _final(partials, x, dinv,
                  conv1_weight, conv1_bias, fc1_weight, fc1_bias,
                  fc2_weight, fc2_bias, fc3_weight, fc3_bias,
                  fc4_weight, fc4_bias)


# speedup vs baseline: 1.2330x; 1.0362x over previous
import functools

import jax
import jax.numpy as jnp
from jax import lax
from jax.experimental import pallas as pl
from jax.experimental.pallas import tpu as pltpu
from jax.experimental.pallas import tpu_sc as plsc

N = 10000
E = 320000
DIN = 128
D2 = 256
DOUT = 128

NC = 2
NS = 16
NW = NC * NS
K = 128
NCH = E // K
CHW = NCH // NW
NEX = NCH - CHW * NW
N_PAD = 10240
RPT = N_PAD // NS

_mesh = plsc.VectorSubcoreMesh(
    core_axis_name="c", subcore_axis_name="s", num_cores=NC, num_subcores=NS)


@functools.partial(
    pl.kernel,
    out_type=jax.ShapeDtypeStruct((NC, N_PAD), jnp.float32),
    mesh=_mesh,
    scratch_types=[
        pltpu.VMEM((2, K), jnp.int32),
        pltpu.VMEM((2, K), jnp.int32),
        pltpu.VMEM((2, K), jnp.int32),
        pltpu.VMEM((2, K), jnp.int32),
        pltpu.VMEM((2, K), jnp.int32),
        pltpu.VMEM((2, K), jnp.int32),
        pltpu.VMEM((K,), jnp.float32),
        pltpu.VMEM((RPT,), jnp.float32),
        pltpu.VMEM_SHARED((N_PAD,), jnp.float32),
        pltpu.SemaphoreType.DMA,
        pltpu.SemaphoreType.DMA,
        pltpu.SemaphoreType.DMA,
        pltpu.SemaphoreType.DMA,
        pltpu.SemaphoreType.DMA,
        pltpu.SemaphoreType.DMA,
        pltpu.SemaphoreType.DMA,
        pltpu.SemaphoreType.DMA,
    ],
)
def _deg_kernel(edge_hbm, out_hbm, e0, e1, e2, e3, e4, e5, ones_v, zbuf, acc,
                si0, si1, si2, si3, si4, si5, ss0, ss1):
    c = lax.axis_index("c")
    s = lax.axis_index("s")
    wid = s * NC + c
    r0 = s * RPT
    idb = (e0, e1, e2, e3, e4, e5)
    semi = (si0, si1, si2, si3, si4, si5)
    sems = (ss0, ss1)
    for j in range(K // 16):
        ones_v[pl.ds(j * 16, 16)] = jnp.ones((16,), jnp.float32)
    for j in range(RPT // 16):
        zbuf[pl.ds(j * 16, 16)] = jnp.zeros((16,), jnp.float32)
    pltpu.sync_copy(zbuf, acc.at[pl.ds(r0, RPT)])
    plsc.subcore_barrier()

    def start_idx(ci, ib):
        pltpu.async_copy(edge_hbm.at[:, pl.ds((wid + 32 * ci) * K, K)],
                         idb[ib], semi[ib])

    def wait_idx(ib):
        pltpu.make_async_copy(edge_hbm.at[:, pl.ds(0, K)], idb[ib],
                              semi[ib]).wait()

    def start_sc(ib, sb):
        pltpu.async_copy(ones_v, acc.at[idb[ib].at[1]], sems[sb], add=True)

    def wait_sc(sb):
        pltpu.make_async_copy(ones_v, acc.at[idb[0].at[1]], sems[sb]).wait()

    for ci in range(3):
        start_idx(ci, ci)
    for ci in range(6):
        wait_idx(ci)
        if ci >= 2:
            wait_sc(ci % 2)
        start_sc(ci, ci % 2)
        start_idx(ci + 3, (ci + 3) % 6)

    def group(g, carry):
        cc = 6 * g
        for j in range(6):
            wait_idx(j)
            wait_sc(j % 2)
            start_sc(j, j % 2)
            start_idx(cc + j + 9, (j + 3) % 6)
        return carry

    lax.fori_loop(0, (CHW - 12) // 6, group, 0)
    cc = CHW - 12
    for j in range(6):
        wait_idx(j)
        wait_sc(j % 2)
        start_sc(j, j % 2)
        if j < 3:
            start_idx(cc + j + 9, (j + 3) % 6)
    wait_sc(0)
    wait_sc(1)

    @pl.when(wid < NEX)
    def _():
        pltpu.sync_copy(edge_hbm.at[:, pl.ds((wid + 32 * CHW) * K, K)], e0)
        pltpu.sync_copy(ones_v, acc.at[e0.at[1]], add=True)

    plsc.subcore_barrier()
    pltpu.sync_copy(acc.at[pl.ds(r0, RPT)], out_hbm.at[c, pl.ds(r0, RPT)])


@functools.partial(
    pl.kernel,
    out_type=jax.ShapeDtypeStruct((NC, N_PAD, DIN), jnp.float32),
    mesh=_mesh,
    scratch_types=[
        pltpu.VMEM((2, K), jnp.int32),
        pltpu.VMEM((2, K), jnp.int32),
        pltpu.VMEM((2, K), jnp.int32),
        pltpu.VMEM((2, K), jnp.int32),
        pltpu.VMEM((K, DIN), jnp.float32),
        pltpu.VMEM((K, DIN), jnp.float32),
        pltpu.VMEM_SHARED((N_PAD, DIN), jnp.float32),
        pltpu.SemaphoreType.DMA,
        pltpu.SemaphoreType.DMA,
        pltpu.SemaphoreType.DMA,
        pltpu.SemaphoreType.DMA,
        pltpu.SemaphoreType.DMA,
        pltpu.SemaphoreType.DMA,
    ],
)
def _scatter_kernel(edge_hbm, y_hbm, out_hbm,
                    e0, e1, e2, e3, rows0, rows1, acc,
                    si0, si1, si2, si3, sg0, sg1):
    c = lax.axis_index("c")
    s = lax.axis_index("s")
    wid = s * NC + c
    r0 = s * RPT
    idb = (e0, e1, e2, e3)
    semi = (si0, si1, si2, si3)
    rows = (rows0, rows1)
    semg = (sg0, sg1)

    def zfill(r, carry):
        for j in range(DIN // 16):
            rows0[r, pl.ds(j * 16, 16)] = jnp.zeros((16,), jnp.float32)
        return carry

    lax.fori_loop(0, K, zfill, 0)
    for t in range(RPT // K):
        pltpu.sync_copy(rows0, acc.at[pl.ds(r0 + t * K, K), :])
    plsc.subcore_barrier()

    def start_idx(ci, ib):
        pltpu.async_copy(edge_hbm.at[:, pl.ds((wid + 32 * ci) * K, K)],
                         idb[ib], semi[ib])

    def wait_idx(ib):
        pltpu.make_async_copy(edge_hbm.at[:, pl.ds(0, K)], idb[ib],
                              semi[ib]).wait()

    def start_gather(ib, rb):
        pltpu.async_copy(y_hbm.at[idb[ib].at[0]], rows[rb], semg[rb])

    def wait_gather(rb):
        pltpu.make_async_copy(y_hbm.at[idb[0].at[0]], rows[rb],
                              semg[rb]).wait()

    def scatter_add(ib, rb):
        pltpu.sync_copy(rows[rb], acc.at[idb[ib].at[1]], add=True)

    start_idx(0, 0)
    start_idx(1, 1)
    wait_idx(0)
    start_gather(0, 0)
    start_idx(2, 2)
    wait_idx(1)
    start_gather(1, 1)
    wait_gather(0)
    scatter_add(0, 0)
    start_idx(3, 3)

    def group(g, carry):
        cc = 4 * g
        for j in range(4):
            ib = (j + 2) % 4
            rb = j % 2
            wait_idx(ib)
            start_gather(ib, rb)
            wait_gather(1 - rb)
            scatter_add((j + 1) % 4, 1 - rb)
            start_idx(cc + j + 4, j)
        return carry

    lax.fori_loop(0, (CHW - 6) // 4, group, 0)
    cc = CHW - 6
    for j in range(4):
        ib = (j + 2) % 4
        rb = j % 2
        wait_idx(ib)
        start_gather(ib, rb)
        wait_gather(1 - rb)
        scatter_add((j + 1) % 4, 1 - rb)
        if j < 2:
            start_idx(cc + j + 4, j)
    wait_gather(1)
    scatter_add(1, 1)

    @pl.when(wid < NEX)
    def _():
        pltpu.sync_copy(edge_hbm.at[:, pl.ds((wid + 32 * CHW) * K, K)], e0)
        pltpu.async_copy(y_hbm.at[e0.at[0]], rows0, sg0).wait()
        pltpu.sync_copy(rows0, acc.at[e0.at[1]], add=True)

    plsc.subcore_barrier()
    pltpu.sync_copy(acc.at[pl.ds(r0, RPT), :],
                    out_hbm.at[c, pl.ds(r0, RPT), :])


def _prep_body(degp_ref, x_ref, y_ref, dinv_ref):
    d1 = degp_ref[0] + degp_ref[1] + 1.0
    dinv1 = lax.rsqrt(d1)
    dinv_ref[...] = dinv1
    dcol = jnp.reshape(dinv1, (N_PAD, 1))
    y_ref[pl.ds(0, N), :] = x_ref[...] * dcol[0:N]
    y_ref[pl.ds(N, N_PAD - N), :] = jnp.zeros((N_PAD - N, DIN), jnp.float32)


def _prep(deg_partials, x):
    return pl.pallas_call(
        _prep_body,
        out_shape=(
            jax.ShapeDtypeStruct((N_PAD, DIN), jnp.float32),
            jax.ShapeDtypeStruct((N_PAD,), jnp.float32),
        ),
    )(deg_partials, x)


_NBLK = 16
_BR = N_PAD // _NBLK


def _final_body(p_ref, x_ref, dinv_ref, w1_ref, b1_ref,
                f1w_ref, f1b_ref, f2w_ref, f2b_ref,
                f3w_ref, f3b_ref, f4w_ref, f4b_ref,
                o1_ref, o2_ref, o3_ref, o4_ref, gacc_ref):
    i = pl.program_id(0)
    dv = jnp.reshape(dinv_ref[pl.ds(i * _BR, _BR)], (_BR, 1))
    sv = dv * (p_ref[0] + p_ref[1]) + (dv * dv) * x_ref[...]
    t = jnp.dot(sv, w1_ref[...], preferred_element_type=jnp.float32)
    t = jnp.maximum(t + b1_ref[...], 0.0)
    rows = i * _BR + lax.broadcasted_iota(jnp.int32, (_BR, 1), 0)
    t = jnp.where(rows < N, t, 0.0)

    @pl.when(i == 0)
    def _():
        gacc_ref[...] = jnp.zeros((1, D2), jnp.float32)

    gacc_ref[...] += jnp.sum(t, axis=0, keepdims=True)

    @pl.when(i == _NBLK - 1)
    def _():
        g = gacc_ref[...] * (1.0 / N)
        dn = (((1,), (1,)), ((), ()))
        o1_ref[...] = jnp.tanh(
            lax.dot_general(g, f1w_ref[...], dn,
                            preferred_element_type=jnp.float32)[0]
            + f1b_ref[...])
        o2_ref[...] = jnp.tanh(
            lax.dot_general(g, f2w_ref[...], dn,
                            preferred_element_type=jnp.float32)[0]
            + f2b_ref[...])
        o3_ref[...] = jnp.tanh(
            lax.dot_general(g, f3w_ref[...], dn,
                            preferred_element_type=jnp.float32)[0]
            + f3b_ref[...])
        o4_ref[...] = jnp.tanh(
            lax.dot_general(g, f4w_ref[...], dn,
                            preferred_element_type=jnp.float32)[0]
            + f4b_ref[...])


def _final(partials, x, dinv, w1, b1, f1w, f1b, f2w, f2b,
           f3w, f3b, f4w, f4b):
    full = lambda shape: pl.BlockSpec(shape, lambda i: (0,) * len(shape))
    return pl.pallas_call(
        _final_body,
        grid=(_NBLK,),
        in_specs=[
            pl.BlockSpec((NC, _BR, DIN), lambda i: (0, i, 0)),
            pl.BlockSpec((_BR, DIN), lambda i: (i, 0)),
            pl.BlockSpec((N_PAD,), lambda i: (0,)),
            full((DIN, D2)), full((D2,)),
            full((D2, D2)), full((D2,)),
            full((D2, D2)), full((D2,)),
            full((DOUT, D2)), full((DOUT,)),
            full((DOUT, D2)), full((DOUT,)),
        ],
        out_specs=[full((D2,)), full((D2,)), full((DOUT,)), full((DOUT,))],
        out_shape=(
            jax.ShapeDtypeStruct((D2,), jnp.float32),
            jax.ShapeDtypeStruct((D2,), jnp.float32),
            jax.ShapeDtypeStruct((DOUT,), jnp.float32),
            jax.ShapeDtypeStruct((DOUT,), jnp.float32),
        ),
        scratch_shapes=[pltpu.VMEM((1, D2), jnp.float32)],
    )(partials, x, dinv, w1, b1, f1w, f1b, f2w, f2b, f3w, f3b, f4w, f4b)


def kernel(x, edge_index, conv1_weight, conv1_bias,
           fc1_weight, fc1_bias, fc2_weight, fc2_bias,
           fc3_weight, fc3_bias, fc4_weight, fc4_bias):
    deg_partials = _deg_kernel(edge_index)
    y, dinv = _prep(deg_partials, x)
    partials = _scatter_kernel(edge_index, y)
    return _final(partials, x, dinv,
                  conv1_weight, conv1_bias, fc1_weight, fc1_bias,
                  fc2_weight, fc2_bias, fc3_weight, fc3_bias,
                  fc4_weight, fc4_bias)
